# baseline (speedup 1.0000x reference)
import functools

import jax
import jax.numpy as jnp
import numpy as np
from jax import lax
from jax.experimental import pallas as pl
from jax.experimental.pallas import tpu as pltpu
from jax.experimental.pallas import tpu_sc as plsc

LEVELS = 16
FEATURES = 8
TABLE_SIZE = 524288
N_POINTS = 1024

_NC = 2
_NS = 16
_NW = _NC * _NS
_CHUNK = (LEVELS * N_POINTS) // _NW
_LANES = 16
_ROW = 128
_PACK = _ROW // FEATURES

_PREF = [min(TABLE_SIZE, -(-(3 * 2 ** (l + 1) + 1) // _PACK) * _PACK)
         for l in range(LEVELS)]
_OFFS = np.concatenate([[0], np.cumsum([p // _PACK for p in _PREF])])
_NROWS = int(_OFFS[-1])


def _sc_gather(x0, x1, x2, offs, packed):
    mesh = plsc.VectorSubcoreMesh(core_axis_name="c", subcore_axis_name="s")

    @functools.partial(
        pl.kernel,
        mesh=mesh,
        compiler_params=pltpu.CompilerParams(needs_layout_passes=False),
        out_type=jax.ShapeDtypeStruct((LEVELS * N_POINTS, _ROW), jnp.float32),
        scratch_types=[
            pltpu.VMEM((_CHUNK,), jnp.float32),
            pltpu.VMEM((_CHUNK,), jnp.float32),
            pltpu.VMEM((_CHUNK,), jnp.float32),
            pltpu.VMEM((_LANES,), jnp.int32),
            pltpu.VMEM((_CHUNK,), jnp.int32),
            pltpu.VMEM((_CHUNK, _ROW), jnp.float32),
        ],
    )
    def k(x0_hbm, x1_hbm, x2_hbm, off_hbm, tab_hbm, out_hbm,
          x0_v, x1_v, x2_v, off_v, idx_v, rows_v):
        wid = lax.axis_index("s") * _NC + lax.axis_index("c")
        base = wid * _CHUNK
        lvl = base // N_POINTS
        j0 = base % N_POINTS
        g = lax.shift_left(jnp.int32(2), lvl).astype(jnp.float32)
        pltpu.sync_copy(x0_hbm.at[pl.ds(j0, _CHUNK)], x0_v)
        pltpu.sync_copy(x1_hbm.at[pl.ds(j0, _CHUNK)], x1_v)
        pltpu.sync_copy(x2_hbm.at[pl.ds(j0, _CHUNK)], x2_v)
        pltpu.sync_copy(off_hbm, off_v)

        iota16 = lax.iota(jnp.int32, _LANES)
        off = jnp.sum(jnp.where(iota16 == lvl, off_v[...], 0))

        @pl.loop(0, _CHUNK, step=_LANES)
        def _(c):
            s = pl.ds(c, _LANES)
            v = (
                (x0_v[s] * g).astype(jnp.int32)
                + (x1_v[s] * g).astype(jnp.int32)
                + (x2_v[s] * g).astype(jnp.int32)
            )
            h = lax.rem(v, jnp.int32(TABLE_SIZE))
            idx_v[s] = off + lax.shift_right_logical(h, 4)

        pltpu.sync_copy(tab_hbm.at[idx_v], rows_v)
        pltpu.sync_copy(rows_v, out_hbm.at[pl.ds(base, _CHUNK)])

    return k(x0, x1, x2, offs, packed)


def _mm_body(x_ref, feat_ref, o_ref):
    lvl = pl.program_id(0)
    g = lax.shift_left(jnp.int32(2), lvl).astype(jnp.float32)
    xs = x_ref[...]
    z = jnp.zeros((N_POINTS, 1), jnp.float32)
    xr = jnp.concatenate(
        [xs[:, 0:1], xs[:, 0:1], xs[:, 1:2], xs[:, 1:2], xs[:, 2:3], xs[:, 2:3], z, z],
        axis=1,
    )
    t = xr * g
    ft = jnp.floor(t)
    f = t - ft
    lane = lax.broadcasted_iota(jnp.int32, (N_POINTS, FEATURES), 1)
    w = jnp.where((lane & 1) == 1, f, 1.0 - f)
    a = w * jnp.where(lane < 6, jnp.float32(1.0 / 3.0), jnp.float32(0.0))

    sv = (ft[:, 0:1] + ft[:, 2:3] + ft[:, 4:5]).astype(jnp.int32)
    sub = sv & (_PACK - 1)
    gidx = sub * FEATURES + lane
    feat = jnp.take_along_axis(feat_ref[0], gidx, axis=1)

    o_ref[...] = lax.dot_general(
        a, feat, (((1,), (1,)), ((), ())), preferred_element_type=jnp.float32
    )


def _interp_matmul(x, feat3):
    return pl.pallas_call(
        _mm_body,
        grid=(LEVELS,),
        in_specs=[
            pl.BlockSpec((N_POINTS, 3), lambda l: (0, 0)),
            pl.BlockSpec((1, N_POINTS, _ROW), lambda l: (l, 0, 0)),
        ],
        out_specs=pl.BlockSpec((N_POINTS, N_POINTS), lambda l: (0, l)),
        out_shape=jax.ShapeDtypeStruct((N_POINTS, LEVELS * N_POINTS), jnp.float32),
    )(x, feat3)


def kernel(x, tables):
    fm = jnp.concatenate(
        [tables[l].T[:, : _PREF[l]] for l in range(LEVELS)], axis=1
    )
    packed = fm.T.reshape(_NROWS, _ROW)
    offs = jnp.asarray(_OFFS[:LEVELS], dtype=jnp.int32)
    wide = _sc_gather(x[:, 0], x[:, 1], x[:, 2], offs, packed)
    feat3 = wide.reshape(LEVELS, N_POINTS, _ROW)
    return _interp_matmul(x, feat3)


# speedup vs baseline: 1.2347x; 1.2347x over previous
import functools

import jax
import jax.numpy as jnp
import numpy as np
from jax import lax
from jax.experimental import pallas as pl
from jax.experimental.pallas import tpu as pltpu
from jax.experimental.pallas import tpu_sc as plsc

LEVELS = 16
FEATURES = 8
TABLE_SIZE = 524288
N_POINTS = 1024

_NC = 2
_NS = 16
_NW = _NC * _NS
_CHUNK = (LEVELS * N_POINTS) // _NW
_LANES = 16
_ROW = 128
_PACK = _ROW // FEATURES

_C = 2048
_PREF = [min(TABLE_SIZE, -(-(3 * 2 ** (l + 1) + 1) // _C) * _C)
         for l in range(LEVELS)]
_OFFS = np.concatenate([[0], np.cumsum([p // _PACK for p in _PREF])])
_NROWS = int(_OFFS[-1])
_CHUNK_LVL = np.array(
    [l for l in range(LEVELS) for _ in range(_PREF[l] // _C)], np.int32)
_CHUNK_POS = np.array(
    [j for l in range(LEVELS) for j in range(_PREF[l] // _C)], np.int32)
_NCHUNKS = len(_CHUNK_LVL)


def _repack_body(lvl_ref, pc_ref, in_ref, out_ref):
    out_ref[...] = jnp.swapaxes(in_ref[0], 0, 1)


def _repack(tt):
    grid_spec = pltpu.PrefetchScalarGridSpec(
        num_scalar_prefetch=2,
        grid=(_NCHUNKS,),
        in_specs=[
            pl.BlockSpec(
                (1, FEATURES, _C),
                lambda i, lvl_ref, pc_ref: (lvl_ref[i], 0, pc_ref[i]),
            ),
        ],
        out_specs=pl.BlockSpec((_C, FEATURES), lambda i, lvl_ref, pc_ref: (i, 0)),
    )
    return pl.pallas_call(
        _repack_body,
        grid_spec=grid_spec,
        out_shape=jax.ShapeDtypeStruct((_NCHUNKS * _C, FEATURES), jnp.float32),
    )(jnp.asarray(_CHUNK_LVL), jnp.asarray(_CHUNK_POS), tt)


def _sc_gather(x0, x1, x2, offs, packed):
    mesh = plsc.VectorSubcoreMesh(core_axis_name="c", subcore_axis_name="s")

    @functools.partial(
        pl.kernel,
        mesh=mesh,
        compiler_params=pltpu.CompilerParams(needs_layout_passes=False),
        out_type=jax.ShapeDtypeStruct((LEVELS * N_POINTS, _ROW), jnp.float32),
        scratch_types=[
            pltpu.VMEM((_CHUNK,), jnp.float32),
            pltpu.VMEM((_CHUNK,), jnp.float32),
            pltpu.VMEM((_CHUNK,), jnp.float32),
            pltpu.VMEM((_LANES,), jnp.int32),
            pltpu.VMEM((_CHUNK,), jnp.int32),
            pltpu.VMEM((_CHUNK, _ROW), jnp.float32),
        ],
    )
    def k(x0_hbm, x1_hbm, x2_hbm, off_hbm, tab_hbm, out_hbm,
          x0_v, x1_v, x2_v, off_v, idx_v, rows_v):
        wid = lax.axis_index("s") * _NC + lax.axis_index("c")
        base = wid * _CHUNK
        lvl = base // N_POINTS
        j0 = base % N_POINTS
        g = lax.shift_left(jnp.int32(2), lvl).astype(jnp.float32)
        pltpu.sync_copy(x0_hbm.at[pl.ds(j0, _CHUNK)], x0_v)
        pltpu.sync_copy(x1_hbm.at[pl.ds(j0, _CHUNK)], x1_v)
        pltpu.sync_copy(x2_hbm.at[pl.ds(j0, _CHUNK)], x2_v)
        pltpu.sync_copy(off_hbm, off_v)

        iota16 = lax.iota(jnp.int32, _LANES)
        off = jnp.sum(jnp.where(iota16 == lvl, off_v[...], 0))

        @pl.loop(0, _CHUNK, step=_LANES)
        def _(c):
            s = pl.ds(c, _LANES)
            v = (
                (x0_v[s] * g).astype(jnp.int32)
                + (x1_v[s] * g).astype(jnp.int32)
                + (x2_v[s] * g).astype(jnp.int32)
            )
            h = lax.rem(v, jnp.int32(TABLE_SIZE))
            idx_v[s] = off + lax.shift_right_logical(h, 4)

        pltpu.sync_copy(tab_hbm.at[idx_v], rows_v)
        pltpu.sync_copy(rows_v, out_hbm.at[pl.ds(base, _CHUNK)])

    return k(x0, x1, x2, offs, packed)


def _mm_body(x_ref, feat_ref, o_ref):
    lvl = pl.program_id(0)
    g = lax.shift_left(jnp.int32(2), lvl).astype(jnp.float32)
    xs = x_ref[...]
    z = jnp.zeros((N_POINTS, 1), jnp.float32)
    xr = jnp.concatenate(
        [xs[:, 0:1], xs[:, 0:1], xs[:, 1:2], xs[:, 1:2], xs[:, 2:3], xs[:, 2:3], z, z],
        axis=1,
    )
    t = xr * g
    ft = jnp.floor(t)
    f = t - ft
    lane = lax.broadcasted_iota(jnp.int32, (N_POINTS, FEATURES), 1)
    w = jnp.where((lane & 1) == 1, f, 1.0 - f)
    a = w * jnp.where(lane < 6, jnp.float32(1.0 / 3.0), jnp.float32(0.0))

    sv = (ft[:, 0:1] + ft[:, 2:3] + ft[:, 4:5]).astype(jnp.int32)
    sub = sv & (_PACK - 1)
    gidx = sub * FEATURES + lane
    feat = jnp.take_along_axis(feat_ref[0], gidx, axis=1)

    o_ref[...] = lax.dot_general(
        a, feat, (((1,), (1,)), ((), ())), preferred_element_type=jnp.float32
    )


def _interp_matmul(x, feat3):
    return pl.pallas_call(
        _mm_body,
        grid=(LEVELS,),
        in_specs=[
            pl.BlockSpec((N_POINTS, 3), lambda l: (0, 0)),
            pl.BlockSpec((1, N_POINTS, _ROW), lambda l: (l, 0, 0)),
        ],
        out_specs=pl.BlockSpec((N_POINTS, N_POINTS), lambda l: (0, l)),
        out_shape=jax.ShapeDtypeStruct((N_POINTS, LEVELS * N_POINTS), jnp.float32),
    )(x, feat3)


def kernel(x, tables):
    tt = tables.transpose(0, 2, 1)
    packed = _repack(tt).reshape(_NROWS, _ROW)
    offs = jnp.asarray(_OFFS[:LEVELS], dtype=jnp.int32)
    wide = _sc_gather(x[:, 0], x[:, 1], x[:, 2], offs, packed)
    feat3 = wide.reshape(LEVELS, N_POINTS, _ROW)
    return _interp_matmul(x, feat3)


# speedup vs baseline: 2.0606x; 1.6689x over previous
import functools

import jax
import jax.numpy as jnp
import numpy as np
from jax import lax
from jax.experimental import pallas as pl
from jax.experimental.pallas import tpu as pltpu
from jax.experimental.pallas import tpu_sc as plsc

LEVELS = 16
FEATURES = 8
TABLE_SIZE = 524288
N_POINTS = 1024

_NC = 2
_NS = 16
_NW = _NC * _NS
_CHUNK = (LEVELS * N_POINTS) // _NW
_LANES = 16
_ROW = 128
_PACK = _ROW // FEATURES

_C = 2048
_PREF = [min(TABLE_SIZE, -(-(3 * 2 ** (l + 1) + 1) // _C) * _C)
         for l in range(LEVELS)]
_OFFS = np.concatenate([[0], np.cumsum([p // _PACK for p in _PREF])])
_NROWS = int(_OFFS[-1])
_CHUNK_LVL = np.array(
    [l for l in range(LEVELS) for _ in range(_PREF[l] // _C)], np.int32)
_CHUNK_POS = np.array(
    [j for l in range(LEVELS) for j in range(_PREF[l] // _C)], np.int32)
_NCHUNKS = len(_CHUNK_LVL)


def _repack_body(lvl_ref, pc_ref, in_ref, out_ref):
    x = in_ref[0]
    out_ref[...] = (
        x.reshape(FEATURES, _C // _PACK, _PACK)
        .swapaxes(0, 1)
        .reshape(_C // _PACK, _ROW)
    )


def _repack(tt):
    grid_spec = pltpu.PrefetchScalarGridSpec(
        num_scalar_prefetch=2,
        grid=(_NCHUNKS,),
        in_specs=[
            pl.BlockSpec(
                (1, FEATURES, _C),
                lambda i, lvl_ref, pc_ref: (lvl_ref[i], 0, pc_ref[i]),
            ),
        ],
        out_specs=pl.BlockSpec(
            (_C // _PACK, _ROW), lambda i, lvl_ref, pc_ref: (i, 0)
        ),
    )
    return pl.pallas_call(
        _repack_body,
        grid_spec=grid_spec,
        out_shape=jax.ShapeDtypeStruct((_NROWS, _ROW), jnp.float32),
    )(jnp.asarray(_CHUNK_LVL), jnp.asarray(_CHUNK_POS), tt)


def _sc_gather(x0, x1, x2, offs, packed):
    mesh = plsc.VectorSubcoreMesh(core_axis_name="c", subcore_axis_name="s")

    @functools.partial(
        pl.kernel,
        mesh=mesh,
        compiler_params=pltpu.CompilerParams(needs_layout_passes=False),
        out_type=jax.ShapeDtypeStruct((LEVELS * N_POINTS, _ROW), jnp.float32),
        scratch_types=[
            pltpu.VMEM((_CHUNK,), jnp.float32),
            pltpu.VMEM((_CHUNK,), jnp.float32),
            pltpu.VMEM((_CHUNK,), jnp.float32),
            pltpu.VMEM((_LANES,), jnp.int32),
            pltpu.VMEM((_CHUNK,), jnp.int32),
            pltpu.VMEM((_CHUNK, _ROW), jnp.float32),
        ],
    )
    def k(x0_hbm, x1_hbm, x2_hbm, off_hbm, tab_hbm, out_hbm,
          x0_v, x1_v, x2_v, off_v, idx_v, rows_v):
        wid = lax.axis_index("s") * _NC + lax.axis_index("c")
        base = wid * _CHUNK
        lvl = base // N_POINTS
        j0 = base % N_POINTS
        g = lax.shift_left(jnp.int32(2), lvl).astype(jnp.float32)
        pltpu.sync_copy(x0_hbm.at[pl.ds(j0, _CHUNK)], x0_v)
        pltpu.sync_copy(x1_hbm.at[pl.ds(j0, _CHUNK)], x1_v)
        pltpu.sync_copy(x2_hbm.at[pl.ds(j0, _CHUNK)], x2_v)
        pltpu.sync_copy(off_hbm, off_v)

        iota16 = lax.iota(jnp.int32, _LANES)
        off = jnp.sum(jnp.where(iota16 == lvl, off_v[...], 0))

        @pl.loop(0, _CHUNK, step=_LANES)
        def _(c):
            s = pl.ds(c, _LANES)
            v = (
                (x0_v[s] * g).astype(jnp.int32)
                + (x1_v[s] * g).astype(jnp.int32)
                + (x2_v[s] * g).astype(jnp.int32)
            )
            h = lax.rem(v, jnp.int32(TABLE_SIZE))
            idx_v[s] = off + lax.shift_right_logical(h, 4)

        pltpu.sync_copy(tab_hbm.at[idx_v], rows_v)
        pltpu.sync_copy(rows_v, out_hbm.at[pl.ds(base, _CHUNK)])

    return k(x0, x1, x2, offs, packed)


def _mm_body(x_ref, feat_ref, o_ref):
    lvl = pl.program_id(0)
    g = lax.shift_left(jnp.int32(2), lvl).astype(jnp.float32)
    xs = x_ref[...]
    z = jnp.zeros((N_POINTS, 1), jnp.float32)
    xr = jnp.concatenate(
        [xs[:, 0:1], xs[:, 0:1], xs[:, 1:2], xs[:, 1:2], xs[:, 2:3], xs[:, 2:3], z, z],
        axis=1,
    )
    t = xr * g
    ft = jnp.floor(t)
    f = t - ft
    lane = lax.broadcasted_iota(jnp.int32, (N_POINTS, FEATURES), 1)
    w = jnp.where((lane & 1) == 1, f, 1.0 - f)
    a = w * jnp.where(lane < 6, jnp.float32(1.0 / 3.0), jnp.float32(0.0))

    sv = (ft[:, 0:1] + ft[:, 2:3] + ft[:, 4:5]).astype(jnp.int32)
    sub = sv & (_PACK - 1)
    gidx = lane * _PACK + sub
    feat = jnp.take_along_axis(feat_ref[0], gidx, axis=1)

    o_ref[...] = lax.dot_general(
        a, feat, (((1,), (1,)), ((), ())), preferred_element_type=jnp.float32
    )


def _interp_matmul(x, feat3):
    return pl.pallas_call(
        _mm_body,
        grid=(LEVELS,),
        in_specs=[
            pl.BlockSpec((N_POINTS, 3), lambda l: (0, 0)),
            pl.BlockSpec((1, N_POINTS, _ROW), lambda l: (l, 0, 0)),
        ],
        out_specs=pl.BlockSpec((N_POINTS, N_POINTS), lambda l: (0, l)),
        out_shape=jax.ShapeDtypeStruct((N_POINTS, LEVELS * N_POINTS), jnp.float32),
    )(x, feat3)


def kernel(x, tables):
    tt = tables.transpose(0, 2, 1)
    packed = _repack(tt)
    offs = jnp.asarray(_OFFS[:LEVELS], dtype=jnp.int32)
    wide = _sc_gather(x[:, 0], x[:, 1], x[:, 2], offs, packed)
    feat3 = wide.reshape(LEVELS, N_POINTS, _ROW)
    return _interp_matmul(x, feat3)


# speedup vs baseline: 2.0642x; 1.0017x over previous
import functools

import jax
import jax.numpy as jnp
import numpy as np
from jax import lax
from jax.experimental import pallas as pl
from jax.experimental.pallas import tpu as pltpu
from jax.experimental.pallas import tpu_sc as plsc

LEVELS = 16
FEATURES = 8
TABLE_SIZE = 524288
N_POINTS = 1024

_NC = 2
_NS = 16
_NW = _NC * _NS
_CHUNK = (LEVELS * N_POINTS) // _NW
_LANES = 16
_ROW = 128
_PACK = _ROW // FEATURES

_C = 2048
_PREF = [min(TABLE_SIZE, -(-(3 * 2 ** (l + 1) + 1) // _C) * _C)
         for l in range(LEVELS)]
_OFFS = np.concatenate([[0], np.cumsum([p // _PACK for p in _PREF])])
_NROWS = int(_OFFS[-1])
_CHUNK_LVL = np.array(
    [l for l in range(LEVELS) for _ in range(_PREF[l] // _C)], np.int32)
_CHUNK_POS = np.array(
    [j for l in range(LEVELS) for j in range(_PREF[l] // _C)], np.int32)
_NCHUNKS = len(_CHUNK_LVL)


def _repack_body(lvl_ref, pc_ref, in_ref, out_ref):
    x = in_ref[0]
    out_ref[...] = (
        x.reshape(FEATURES, _C // _PACK, _PACK)
        .swapaxes(0, 1)
        .reshape(_C // _PACK, _ROW)
    )


def _repack(tt):
    grid_spec = pltpu.PrefetchScalarGridSpec(
        num_scalar_prefetch=2,
        grid=(_NCHUNKS,),
        in_specs=[
            pl.BlockSpec(
                (1, FEATURES, _C),
                lambda i, lvl_ref, pc_ref: (lvl_ref[i], 0, pc_ref[i]),
            ),
        ],
        out_specs=pl.BlockSpec(
            (_C // _PACK, _ROW), lambda i, lvl_ref, pc_ref: (i, 0)
        ),
    )
    return pl.pallas_call(
        _repack_body,
        grid_spec=grid_spec,
        out_shape=jax.ShapeDtypeStruct((_NROWS, _ROW), jnp.float32),
        compiler_params=pltpu.CompilerParams(
            dimension_semantics=("parallel",)
        ),
    )(jnp.asarray(_CHUNK_LVL), jnp.asarray(_CHUNK_POS), tt)


def _sc_gather(x0, x1, x2, offs, packed):
    mesh = plsc.VectorSubcoreMesh(core_axis_name="c", subcore_axis_name="s")

    @functools.partial(
        pl.kernel,
        mesh=mesh,
        compiler_params=pltpu.CompilerParams(needs_layout_passes=False),
        out_type=jax.ShapeDtypeStruct((LEVELS * N_POINTS, _ROW), jnp.float32),
        scratch_types=[
            pltpu.VMEM((_CHUNK,), jnp.float32),
            pltpu.VMEM((_CHUNK,), jnp.float32),
            pltpu.VMEM((_CHUNK,), jnp.float32),
            pltpu.VMEM((_LANES,), jnp.int32),
            pltpu.VMEM((_CHUNK,), jnp.int32),
            pltpu.VMEM((_CHUNK, _ROW), jnp.float32),
        ],
    )
    def k(x0_hbm, x1_hbm, x2_hbm, off_hbm, tab_hbm, out_hbm,
          x0_v, x1_v, x2_v, off_v, idx_v, rows_v):
        wid = lax.axis_index("s") * _NC + lax.axis_index("c")
        base = wid * _CHUNK
        lvl = base // N_POINTS
        j0 = base % N_POINTS
        g = lax.shift_left(jnp.int32(2), lvl).astype(jnp.float32)
        pltpu.sync_copy(x0_hbm.at[pl.ds(j0, _CHUNK)], x0_v)
        pltpu.sync_copy(x1_hbm.at[pl.ds(j0, _CHUNK)], x1_v)
        pltpu.sync_copy(x2_hbm.at[pl.ds(j0, _CHUNK)], x2_v)
        pltpu.sync_copy(off_hbm, off_v)

        iota16 = lax.iota(jnp.int32, _LANES)
        off = jnp.sum(jnp.where(iota16 == lvl, off_v[...], 0))

        @pl.loop(0, _CHUNK, step=_LANES)
        def _(c):
            s = pl.ds(c, _LANES)
            v = (
                (x0_v[s] * g).astype(jnp.int32)
                + (x1_v[s] * g).astype(jnp.int32)
                + (x2_v[s] * g).astype(jnp.int32)
            )
            h = lax.rem(v, jnp.int32(TABLE_SIZE))
            idx_v[s] = off + lax.shift_right_logical(h, 4)

        pltpu.sync_copy(tab_hbm.at[idx_v], rows_v)
        pltpu.sync_copy(rows_v, out_hbm.at[pl.ds(base, _CHUNK)])

    return k(x0, x1, x2, offs, packed)


def _mm_body(x_ref, feat_ref, o_ref):
    lvl = pl.program_id(0)
    g = lax.shift_left(jnp.int32(2), lvl).astype(jnp.float32)
    xs = x_ref[...]
    z = jnp.zeros((N_POINTS, 1), jnp.float32)
    xr = jnp.concatenate(
        [xs[:, 0:1], xs[:, 0:1], xs[:, 1:2], xs[:, 1:2], xs[:, 2:3], xs[:, 2:3], z, z],
        axis=1,
    )
    t = xr * g
    ft = jnp.floor(t)
    f = t - ft
    lane = lax.broadcasted_iota(jnp.int32, (N_POINTS, FEATURES), 1)
    w = jnp.where((lane & 1) == 1, f, 1.0 - f)
    a = w * jnp.where(lane < 6, jnp.float32(1.0 / 3.0), jnp.float32(0.0))

    sv = (ft[:, 0:1] + ft[:, 2:3] + ft[:, 4:5]).astype(jnp.int32)
    sub = sv & (_PACK - 1)
    gidx = lane * _PACK + sub
    feat = jnp.take_along_axis(feat_ref[0], gidx, axis=1)

    o_ref[...] = lax.dot_general(
        a, feat, (((1,), (1,)), ((), ())), preferred_element_type=jnp.float32
    )


def _interp_matmul(x, feat3):
    return pl.pallas_call(
        _mm_body,
        grid=(LEVELS,),
        in_specs=[
            pl.BlockSpec((N_POINTS, 3), lambda l: (0, 0)),
            pl.BlockSpec((1, N_POINTS, _ROW), lambda l: (l, 0, 0)),
        ],
        out_specs=pl.BlockSpec((N_POINTS, N_POINTS), lambda l: (0, l)),
        out_shape=jax.ShapeDtypeStruct((N_POINTS, LEVELS * N_POINTS), jnp.float32),
        compiler_params=pltpu.CompilerParams(
            dimension_semantics=("parallel",)
        ),
    )(x, feat3)


def kernel(x, tables):
    tt = tables.transpose(0, 2, 1)
    packed = _repack(tt)
    offs = jnp.asarray(_OFFS[:LEVELS], dtype=jnp.int32)
    wide = _sc_gather(x[:, 0], x[:, 1], x[:, 2], offs, packed)
    feat3 = wide.reshape(LEVELS, N_POINTS, _ROW)
    return _interp_matmul(x, feat3)
